# Initial kernel scaffold; baseline (speedup 1.0000x reference)
import functools

import jax
import jax.numpy as jnp
from jax import lax
from jax.experimental import pallas as pl
from jax.experimental.pallas import tpu as pltpu
from jax.experimental.pallas import tpu_sc as plsc

_VOCAB = 1_000_000
_D = 64
_NC, _NS, _L = 2, 16, 16
_NW = _NC * _NS
_BT = 16384 * 50
_BPW = _BT // _NW
_C = 512
_NCHUNK = _BPW // _C
_NPAIR = _NCHUNK // 2

_mesh = plsc.VectorSubcoreMesh(core_axis_name="c", subcore_axis_name="s")


@functools.partial(
    pl.kernel,
    mesh=_mesh,
    out_type=jax.ShapeDtypeStruct((_BT, _D), jnp.float32),
    scratch_types=[
        pltpu.VMEM((_BPW,), jnp.int32),
        pltpu.VMEM((_C, _D), jnp.float32),
        pltpu.VMEM((_C, _D), jnp.float32),
        pltpu.SemaphoreType.DMA,
        pltpu.SemaphoreType.DMA,
        pltpu.SemaphoreType.DMA,
        pltpu.SemaphoreType.DMA,
    ],
)
def _emb_lookup(token_hbm, table_hbm, out_hbm, idx_v, buf0, buf1, g0, g1, s0, s1):
    wid = lax.axis_index("s") * _NC + lax.axis_index("c")
    base = wid * _BPW
    pltpu.sync_copy(token_hbm.at[pl.ds(base, _BPW)], idx_v)

    def clamp_chunk(c):
        def cb(j, _):
            s = c * _C + j * _L
            v = idx_v[pl.ds(s, _L)]
            idx_v[pl.ds(s, _L)] = jnp.minimum(jnp.maximum(v, 0), _VOCAB - 1)
            return 0
        lax.fori_loop(0, _C // _L, cb, 0)

    def fire_gather(c, buf, sem):
        pltpu.async_copy(table_hbm.at[idx_v.at[pl.ds(c * _C, _C)]], buf, sem)

    def wait_gather(buf, sem):
        pltpu.make_async_copy(table_hbm.at[idx_v.at[pl.ds(0, _C)]], buf, sem).wait()

    def fire_store(c, buf, sem):
        pltpu.async_copy(buf, out_hbm.at[pl.ds(base + c * _C, _C)], sem)

    def wait_store(buf, sem):
        pltpu.make_async_copy(buf, out_hbm.at[pl.ds(base, _C)], sem).wait()

    clamp_chunk(0)
    fire_gather(0, buf0, g0)

    def body(i, _):
        c0 = 2 * i
        c1 = c0 + 1
        wait_gather(buf0, g0)
        fire_store(c0, buf0, s0)
        clamp_chunk(c1)

        @pl.when(i > 0)
        def _wait_prev_odd_store():
            wait_store(buf1, s1)

        fire_gather(c1, buf1, g1)
        wait_gather(buf1, g1)
        fire_store(c1, buf1, s1)
        wait_store(buf0, s0)

        @pl.when(i < _NPAIR - 1)
        def _prefetch_next_even():
            clamp_chunk(c0 + 2)
            fire_gather(c0 + 2, buf0, g0)

        return 0

    lax.fori_loop(0, _NPAIR, body, 0)
    wait_store(buf1, s1)


def kernel(token, table):
    out = _emb_lookup(token.reshape(-1), table)
    return out.reshape(token.shape + (_D,))


# speedup vs baseline: 1.8663x; 1.8663x over previous
import functools

import jax
import jax.numpy as jnp
from jax import lax
from jax.experimental import pallas as pl
from jax.experimental.pallas import tpu as pltpu
from jax.experimental.pallas import tpu_sc as plsc

_VOCAB = 1_000_000
_D = 64
_NC, _NS, _L = 2, 16, 16
_NW = _NC * _NS
_BT = 16384 * 50
_BPW = _BT // _NW
_C = 512
_NCHUNK = _BPW // _C
_NPAIR = _NCHUNK // 2

_mesh = plsc.VectorSubcoreMesh(core_axis_name="c", subcore_axis_name="s")


@functools.partial(
    pl.kernel,
    mesh=_mesh,
    out_type=jax.ShapeDtypeStruct((_BT, _D), jnp.float32),
    compiler_params=pltpu.CompilerParams(use_tc_tiling_on_sc=False),
    scratch_types=[
        pltpu.VMEM((_BPW,), jnp.int32),
        pltpu.VMEM((_C, _D), jnp.float32),
        pltpu.VMEM((_C, _D), jnp.float32),
        pltpu.SemaphoreType.DMA,
        pltpu.SemaphoreType.DMA,
        pltpu.SemaphoreType.DMA,
        pltpu.SemaphoreType.DMA,
    ],
)
def _emb_lookup(token_hbm, table_hbm, out_hbm, idx_v, buf0, buf1, g0, g1, s0, s1):
    wid = lax.axis_index("s") * _NC + lax.axis_index("c")
    base = wid * _BPW
    pltpu.sync_copy(token_hbm.at[pl.ds(base, _BPW)], idx_v)

    def clamp_chunk(c):
        def cb(j, _):
            s = c * _C + j * _L
            v = idx_v[pl.ds(s, _L)]
            idx_v[pl.ds(s, _L)] = jnp.minimum(jnp.maximum(v, 0), _VOCAB - 1)
            return 0
        lax.fori_loop(0, _C // _L, cb, 0)

    def fire_gather(c, buf, sem):
        pltpu.async_copy(table_hbm.at[idx_v.at[pl.ds(c * _C, _C)]], buf, sem)

    def wait_gather(buf, sem):
        pltpu.make_async_copy(table_hbm.at[idx_v.at[pl.ds(0, _C)]], buf, sem).wait()

    def fire_store(c, buf, sem):
        pltpu.async_copy(buf, out_hbm.at[pl.ds(base + c * _C, _C)], sem)

    def wait_store(buf, sem):
        pltpu.make_async_copy(buf, out_hbm.at[pl.ds(base, _C)], sem).wait()

    clamp_chunk(0)
    fire_gather(0, buf0, g0)

    def body(i, _):
        c0 = 2 * i
        c1 = c0 + 1
        wait_gather(buf0, g0)
        fire_store(c0, buf0, s0)
        clamp_chunk(c1)

        @pl.when(i > 0)
        def _wait_prev_odd_store():
            wait_store(buf1, s1)

        fire_gather(c1, buf1, g1)
        wait_gather(buf1, g1)
        fire_store(c1, buf1, s1)
        wait_store(buf0, s0)

        @pl.when(i < _NPAIR - 1)
        def _prefetch_next_even():
            clamp_chunk(c0 + 2)
            fire_gather(c0 + 2, buf0, g0)

        return 0

    lax.fori_loop(0, _NPAIR, body, 0)
    wait_store(buf1, s1)


def kernel(token, table):
    out = _emb_lookup(token.reshape(-1), table)
    return out.reshape(token.shape + (_D,))


# speedup vs baseline: 1.8698x; 1.0019x over previous
import functools

import jax
import jax.numpy as jnp
from jax import lax
from jax.experimental import pallas as pl
from jax.experimental.pallas import tpu as pltpu
from jax.experimental.pallas import tpu_sc as plsc

_VOCAB = 1_000_000
_D = 64
_NC, _NS, _L = 2, 16, 16
_NW = _NC * _NS
_BT = 16384 * 50
_BPW = _BT // _NW
_C = 400
_NCHUNK = _BPW // _C
_NBUF = 4
_NGRP = _NCHUNK // _NBUF

_mesh = plsc.VectorSubcoreMesh(core_axis_name="c", subcore_axis_name="s")


@functools.partial(
    pl.kernel,
    mesh=_mesh,
    out_type=jax.ShapeDtypeStruct((_BT, _D), jnp.float32),
    compiler_params=pltpu.CompilerParams(use_tc_tiling_on_sc=False),
    scratch_types=[
        pltpu.VMEM((_BPW,), jnp.int32),
        pltpu.VMEM((_C, _D), jnp.float32),
        pltpu.VMEM((_C, _D), jnp.float32),
        pltpu.VMEM((_C, _D), jnp.float32),
        pltpu.VMEM((_C, _D), jnp.float32),
        pltpu.SemaphoreType.DMA,
        pltpu.SemaphoreType.DMA,
        pltpu.SemaphoreType.DMA,
        pltpu.SemaphoreType.DMA,
        pltpu.SemaphoreType.DMA,
        pltpu.SemaphoreType.DMA,
        pltpu.SemaphoreType.DMA,
        pltpu.SemaphoreType.DMA,
    ],
)
def _emb_lookup(token_hbm, table_hbm, out_hbm, idx_v,
                b0, b1, b2, b3, g0, g1, g2, g3, s0, s1, s2, s3):
    bufs = (b0, b1, b2, b3)
    gsem = (g0, g1, g2, g3)
    ssem = (s0, s1, s2, s3)
    wid = lax.axis_index("s") * _NC + lax.axis_index("c")
    base = wid * _BPW
    pltpu.sync_copy(token_hbm.at[pl.ds(base, _BPW)], idx_v)

    def clamp_span(lo, hi):
        def cb(j, _):
            s = j * _L
            v = idx_v[pl.ds(s, _L)]
            idx_v[pl.ds(s, _L)] = jnp.minimum(jnp.maximum(v, 0), _VOCAB - 1)
            return 0
        lax.fori_loop(lo // _L, hi // _L, cb, 0)

    def fire_gather(c, buf, sem):
        pltpu.async_copy(table_hbm.at[idx_v.at[pl.ds(c * _C, _C)]], buf, sem)

    def wait_gather(buf, sem):
        pltpu.make_async_copy(table_hbm.at[idx_v.at[pl.ds(0, _C)]], buf, sem).wait()

    def fire_store(c, buf, sem):
        pltpu.async_copy(buf, out_hbm.at[pl.ds(base + c * _C, _C)], sem)

    def wait_store(buf, sem):
        pltpu.make_async_copy(buf, out_hbm.at[pl.ds(base, _C)], sem).wait()

    clamp_span(0, (_NBUF - 1) * _C)
    for b in range(_NBUF - 1):
        fire_gather(b, bufs[b], gsem[b])
    clamp_span((_NBUF - 1) * _C, _BPW)

    def body(g, _):
        i0 = g * _NBUF
        for b in range(_NBUF):
            i = i0 + b
            b2 = (b + _NBUF - 1) % _NBUF
            ahead = i + _NBUF - 1

            @pl.when(jnp.logical_and(ahead < _NCHUNK, i >= 1))
            def _drain_prev_store():
                wait_store(bufs[b2], ssem[b2])

            @pl.when(ahead < _NCHUNK)
            def _launch_ahead():
                fire_gather(ahead, bufs[b2], gsem[b2])

            wait_gather(bufs[b], gsem[b])
            fire_store(i, bufs[b], ssem[b])
        return 0

    lax.fori_loop(0, _NGRP, body, 0)
    for b in range(_NBUF):
        wait_store(bufs[b], ssem[b])


def kernel(token, table):
    out = _emb_lookup(token.reshape(-1), table)
    return out.reshape(token.shape + (_D,))
